# baseline (speedup 1.0000x reference)
import functools

import jax
import jax.numpy as jnp
from jax import lax
from jax.experimental import pallas as pl
from jax.experimental.pallas import tpu as pltpu
from jax.experimental.pallas import tpu_sc as plsc

D = 32
B = 16384
NC, NS = 2, 16
NW = NC * NS
B_PER_W = B // NW
GRP = 16
HALF = B_PER_W // 2
N_GRP_H = HALF // GRP


def _gather_body(table_hbm, idx_hbm, out_hbm, idx_v, rows_v, sem_a, sem_b, sem_s):
    wid = lax.axis_index("s") * NC + lax.axis_index("c")
    base = wid * B_PER_W
    pltpu.sync_copy(idx_hbm.at[pl.ds(base, B_PER_W)], idx_v)
    lane = jnp.arange(GRP, dtype=jnp.int32)

    def fire_half(h, sem_h):
        def group(g, carry):
            vec = idx_v[pl.ds(h * HALF + g * GRP, GRP)]
            for k in range(GRP):
                row = jnp.max(jnp.where(lane == k, vec, 0), axis=0)
                pltpu.async_copy(
                    table_hbm.at[pl.ds(row, 1)],
                    rows_v.at[pl.ds(h * HALF + g * GRP + k, 1)],
                    sem_h,
                )
            return carry

        lax.fori_loop(0, N_GRP_H, group, 0)

    fire_half(0, sem_a)
    fire_half(1, sem_b)
    pltpu.make_async_copy(
        table_hbm.at[pl.ds(0, HALF)], rows_v.at[pl.ds(0, HALF)], sem_a
    ).wait()
    pltpu.async_copy(
        rows_v.at[pl.ds(0, HALF)], out_hbm.at[pl.ds(base, HALF)], sem_s
    )
    pltpu.make_async_copy(
        table_hbm.at[pl.ds(0, HALF)], rows_v.at[pl.ds(HALF, HALF)], sem_b
    ).wait()
    pltpu.async_copy(
        rows_v.at[pl.ds(HALF, HALF)], out_hbm.at[pl.ds(base + HALF, HALF)], sem_s
    )
    pltpu.make_async_copy(rows_v, out_hbm.at[pl.ds(base, B_PER_W)], sem_s).wait()


@jax.jit
def kernel(process_indices, table):
    idx = process_indices.astype(jnp.int32)
    mesh = plsc.VectorSubcoreMesh(core_axis_name="c", subcore_axis_name="s")
    k = functools.partial(
        pl.kernel,
        mesh=mesh,
        out_type=jax.ShapeDtypeStruct((B, D), jnp.float32),
        scratch_types=[
            pltpu.VMEM((B_PER_W,), jnp.int32),
            pltpu.VMEM((B_PER_W, D), jnp.float32),
            pltpu.SemaphoreType.DMA,
            pltpu.SemaphoreType.DMA,
            pltpu.SemaphoreType.DMA,
        ],
        compiler_params=pltpu.CompilerParams(
            use_tc_tiling_on_sc=True, needs_layout_passes=False
        ),
    )(_gather_body)
    return k(table, idx)
